# baseline (speedup 1.0000x reference)
import functools

import jax
import jax.numpy as jnp
from jax import lax
from jax.experimental import pallas as pl
from jax.experimental.pallas import tpu as pltpu
from jax.experimental.pallas import tpu_sc as plsc

F32 = jnp.float32
I32 = jnp.int32

_B, _T = 1024, 50
_P = 25
_E = 32
_IMG = _P * _E
_TAIL = 34
_ROW = _IMG + _TAIL
_CW = 64
_NW = 32
_BPW = _B // _NW
_BB = 64
_M = _BB * _T


def _tc_setup_body(obs_ref, dir_ref, act_ref, rew_ref, done_ref, tile_ref,
                   color_ref, atab_ref, w_ref, b_ref, selt_ref, selc_ref,
                   combo_ref, fused_ref):
    x = obs_ref[...].reshape(_M, 2 * _P).astype(F32)
    t = jnp.dot(x, selt_ref[...], preferred_element_type=F32)
    c = jnp.dot(x, selc_ref[...], preferred_element_type=F32)
    combo_ref[:, :, 0:_P] = (t * 16.0 + c).reshape(_BB, _T, _P)

    de = jnp.dot(dir_ref[...].reshape(_M, 4), w_ref[...],
                 preferred_element_type=F32) + b_ref[...]
    a = act_ref[...].reshape(_M, 1)
    oh = (a == lax.broadcasted_iota(I32, (_M, 16), 1)).astype(F32)
    ae = jnp.dot(oh, atab_ref[...], preferred_element_type=F32)
    combo_ref[:, :, _P:_P + 16] = de.reshape(_BB, _T, 16)
    combo_ref[:, :, _P + 16:_P + 32] = ae.reshape(_BB, _T, 16)
    combo_ref[:, :, _P + 32:_P + 33] = rew_ref[...]
    combo_ref[:, :, _P + 33:_P + 34] = done_ref[...]
    combo_ref[:, :, _P + 34:] = jnp.zeros((_BB, _T, _CW - _P - _TAIL), F32)

    r = lax.broadcasted_iota(I32, (256, 16), 0)
    k = lax.broadcasted_iota(I32, (256, 16), 1)
    oht = ((r // 16) == k).astype(F32)
    ohc = ((r % 16) == k).astype(F32)
    fused_ref[:, 0:16] = jnp.dot(oht, tile_ref[...], preferred_element_type=F32)
    fused_ref[:, 16:32] = jnp.dot(ohc, color_ref[...], preferred_element_type=F32)


def _tc_setup(obs3, obs_dir, prev_action, prev_reward, done, tile_table,
              color_table, action_table, dir_W, dir_b, selt, selc):
    grid = _B // _BB
    bspec3 = lambda w: pl.BlockSpec((_BB, _T, w), lambda i: (i, 0, 0))
    fullspec = lambda s: pl.BlockSpec(s, lambda i: tuple(0 for _ in s))
    return pl.pallas_call(
        _tc_setup_body,
        grid=(grid,),
        in_specs=[
            bspec3(2 * _P),
            bspec3(4),
            bspec3(1),
            bspec3(1),
            bspec3(1),
            fullspec((16, 16)),
            fullspec((16, 16)),
            fullspec((16, 16)),
            fullspec((4, 16)),
            fullspec((1, 16)),
            fullspec((2 * _P, _P)),
            fullspec((2 * _P, _P)),
        ],
        out_specs=[
            bspec3(_CW),
            fullspec((256, 32)),
        ],
        out_shape=[
            jax.ShapeDtypeStruct((_B, _T, _CW), F32),
            jax.ShapeDtypeStruct((256, 32), F32),
        ],
    )(obs3, obs_dir, prev_action, prev_reward, done, tile_table, color_table,
      action_table, dir_W, dir_b, selt, selc)


@functools.partial(
    pl.kernel,
    mesh=plsc.VectorSubcoreMesh(core_axis_name="c", subcore_axis_name="s"),
    compiler_params=pltpu.CompilerParams(use_tc_tiling_on_sc=True,
                                         needs_layout_passes=False),
    out_type=jax.ShapeDtypeStruct((_B, _T, _ROW), F32),
    scratch_types=[
        pltpu.VMEM((256 * _E,), F32),
        pltpu.VMEM((_T, _CW), F32),
        pltpu.VMEM((_T, _CW), F32),
        pltpu.VMEM((_T, _ROW), F32),
        pltpu.VMEM((_T, _ROW), F32),
        pltpu.SemaphoreType.DMA,
        pltpu.SemaphoreType.DMA,
    ],
)
def _sc_encode(combo_hbm, fused_hbm, out_hbm,
               fused_v, combo0, combo1, row0, row1, isem, osem):
    wid = lax.axis_index("s") * 2 + lax.axis_index("c")
    b0 = wid * _BPW
    pltpu.sync_copy(fused_hbm, fused_v)
    pltpu.async_copy(combo_hbm.at[b0], combo0, isem)

    iota = lax.iota(I32, 16)

    def assemble(combo_v, rowbuf):
        def body(r, c):
            pv = combo_v[r, pl.ds(0, 16)].astype(I32) * _E
            qv = combo_v[r, pl.ds(9, 16)].astype(I32) * _E
            for p in range(_P):
                src, lane = (pv, p) if p < 16 else (qv, p - 9)
                offv = lax.gather(
                    src, jnp.full((16, 1), lane, I32),
                    lax.GatherDimensionNumbers(offset_dims=(),
                                               collapsed_slice_dims=(0,),
                                               start_index_map=(0,)),
                    (1,), mode=lax.GatherScatterMode.PROMISE_IN_BOUNDS)
                ia = offv + iota
                rowbuf[r, pl.ds(_E * p, 16)] = plsc.load_gather(fused_v, [ia])
                rowbuf[r, pl.ds(_E * p + 16, 16)] = plsc.load_gather(
                    fused_v, [ia + 16])
            rowbuf[r, pl.ds(_IMG, 16)] = combo_v[r, pl.ds(_P, 16)]
            rowbuf[r, pl.ds(_IMG + 16, 16)] = combo_v[r, pl.ds(_P + 16, 16)]
            rowbuf[r, pl.ds(_IMG + 18, 16)] = combo_v[r, pl.ds(_P + 18, 16)]
            return c

        lax.fori_loop(0, _T, body, 0)

    def stage(j, b, combo_v, combo_n, rowbuf):
        pltpu.make_async_copy(combo_hbm.at[b], combo_v, isem).wait()

        @pl.when(b + 1 < b0 + _BPW)
        def _():
            pltpu.async_copy(combo_hbm.at[b + 1], combo_n, isem)

        @pl.when(j > 1)
        def _():
            pltpu.make_async_copy(rowbuf, out_hbm.at[b], osem).wait()

        assemble(combo_v, rowbuf)
        pltpu.async_copy(rowbuf, out_hbm.at[b], osem)

    def pair_body(j, carry):
        b = b0 + 2 * j
        stage(2 * j, b, combo0, combo1, row0)
        stage(2 * j + 1, b + 1, combo1, combo0, row1)
        return carry

    lax.fori_loop(0, _BPW // 2, pair_body, 0)
    pltpu.make_async_copy(row0, out_hbm.at[b0], osem).wait()
    pltpu.make_async_copy(row1, out_hbm.at[b0], osem).wait()


def kernel(obs_img, obs_dir, prev_action, prev_reward, done, tile_table,
           color_table, action_table, dir_W, dir_b):
    obs3 = obs_img.reshape(_B, _T, 2 * _P).astype(I32)
    ii = jnp.arange(2 * _P)[:, None]
    jj = jnp.arange(_P)[None, :]
    selt = (ii == 2 * jj).astype(F32)
    selc = (ii == 2 * jj + 1).astype(F32)
    combo, fused = _tc_setup(
        obs3, obs_dir, prev_action.reshape(_B, _T, 1).astype(I32),
        prev_reward.reshape(_B, _T, 1), done.reshape(_B, _T, 1),
        tile_table, color_table, action_table, dir_W, dir_b.reshape(1, 16),
        selt, selc)
    return _sc_encode(combo, fused.reshape(-1))


# speedup vs baseline: 1.3853x; 1.3853x over previous
import functools

import jax
import jax.numpy as jnp
from jax import lax
from jax.experimental import pallas as pl
from jax.experimental.pallas import tpu as pltpu
from jax.experimental.pallas import tpu_sc as plsc

F32 = jnp.float32
I32 = jnp.int32

_B, _T = 1024, 50
_P = 25
_E = 32
_IMG = _P * _E
_TAIL = 34
_ROW = _IMG + _TAIL
_CW = 64
_NW = 32
_BPW = _B // _NW
_BB = 64
_M = _BB * _T


def _tc_setup_body(obs_ref, dir_ref, act_ref, rew_ref, done_ref, tile_ref,
                   color_ref, atab_ref, w_ref, b_ref, selt_ref, selc_ref,
                   combo_ref, fused_ref):
    x = obs_ref[...].reshape(_M, 2 * _P).astype(F32)
    t = jnp.dot(x, selt_ref[...], preferred_element_type=F32)
    c = jnp.dot(x, selc_ref[...], preferred_element_type=F32)
    combo_ref[:, :, 0:_P] = (t * 16.0 + c).reshape(_BB, _T, _P)

    de = jnp.dot(dir_ref[...].reshape(_M, 4), w_ref[...],
                 preferred_element_type=F32) + b_ref[...]
    a = act_ref[...].reshape(_M, 1)
    oh = (a == lax.broadcasted_iota(I32, (_M, 16), 1)).astype(F32)
    ae = jnp.dot(oh, atab_ref[...], preferred_element_type=F32)
    combo_ref[:, :, _P:_P + 16] = de.reshape(_BB, _T, 16)
    combo_ref[:, :, _P + 16:_P + 32] = ae.reshape(_BB, _T, 16)
    combo_ref[:, :, _P + 32:_P + 33] = rew_ref[...]
    combo_ref[:, :, _P + 33:_P + 34] = done_ref[...]
    combo_ref[:, :, _P + 34:] = jnp.zeros((_BB, _T, _CW - _P - _TAIL), F32)

    r = lax.broadcasted_iota(I32, (256, 16), 0)
    k = lax.broadcasted_iota(I32, (256, 16), 1)
    oht = ((r // 16) == k).astype(F32)
    ohc = ((r % 16) == k).astype(F32)
    fused_ref[:, 0:16] = jnp.dot(oht, tile_ref[...], preferred_element_type=F32)
    fused_ref[:, 16:32] = jnp.dot(ohc, color_ref[...], preferred_element_type=F32)


def _tc_setup(obs3, obs_dir, prev_action, prev_reward, done, tile_table,
              color_table, action_table, dir_W, dir_b, selt, selc):
    grid = _B // _BB
    bspec3 = lambda w: pl.BlockSpec((_BB, _T, w), lambda i: (i, 0, 0))
    fullspec = lambda s: pl.BlockSpec(s, lambda i: tuple(0 for _ in s))
    return pl.pallas_call(
        _tc_setup_body,
        grid=(grid,),
        in_specs=[
            bspec3(2 * _P),
            bspec3(4),
            bspec3(1),
            bspec3(1),
            bspec3(1),
            fullspec((16, 16)),
            fullspec((16, 16)),
            fullspec((16, 16)),
            fullspec((4, 16)),
            fullspec((1, 16)),
            fullspec((2 * _P, _P)),
            fullspec((2 * _P, _P)),
        ],
        out_specs=[
            bspec3(_CW),
            fullspec((256, 32)),
        ],
        out_shape=[
            jax.ShapeDtypeStruct((_B, _T, _CW), F32),
            jax.ShapeDtypeStruct((256, 32), F32),
        ],
    )(obs3, obs_dir, prev_action, prev_reward, done, tile_table, color_table,
      action_table, dir_W, dir_b, selt, selc)


@functools.partial(
    pl.kernel,
    mesh=plsc.VectorSubcoreMesh(core_axis_name="c", subcore_axis_name="s"),
    compiler_params=pltpu.CompilerParams(use_tc_tiling_on_sc=True,
                                         needs_layout_passes=False),
    out_type=jax.ShapeDtypeStruct((_B, _T, _ROW), F32),
    scratch_types=[
        pltpu.VMEM((256 * _E,), F32),
        pltpu.VMEM((_T, _CW), F32),
        pltpu.VMEM((_T, _CW), F32),
        pltpu.VMEM((_T, _ROW), F32),
        pltpu.VMEM((_T, _ROW), F32),
        pltpu.SemaphoreType.DMA,
        pltpu.SemaphoreType.DMA,
    ],
)
def _sc_encode(combo_hbm, fused_hbm, out_hbm,
               fused_v, combo0, combo1, row0, row1, isem, osem):
    wid = lax.axis_index("s") * 2 + lax.axis_index("c")
    b0 = wid * _BPW
    pltpu.sync_copy(fused_hbm, fused_v)
    pltpu.async_copy(combo_hbm.at[b0], combo0, isem)

    iota = lax.iota(I32, 16)

    def assemble(combo_v, rowbuf):
        dnums = lax.GatherDimensionNumbers(offset_dims=(),
                                           collapsed_slice_dims=(0,),
                                           start_index_map=(0,))

        def body(r, c):
            pv = combo_v[r, pl.ds(0, 16)].astype(I32) * _E
            qv = combo_v[r, pl.ds(9, 16)].astype(I32) * _E
            for p0 in range(0, _P, 8):
                vals = []
                for p in range(p0, min(p0 + 8, _P)):
                    src, lane = (pv, p) if p < 16 else (qv, p - 9)
                    offv = lax.gather(
                        src, jnp.full((16, 1), lane, I32), dnums, (1,),
                        mode=lax.GatherScatterMode.PROMISE_IN_BOUNDS)
                    ia = offv + iota
                    vals.append(plsc.load_gather(fused_v, [ia]))
                    vals.append(plsc.load_gather(fused_v, [ia + 16]))
                for k, v in enumerate(vals):
                    rowbuf[r, pl.ds(_E * p0 + 16 * k, 16)] = v
            rowbuf[r, pl.ds(_IMG, 16)] = combo_v[r, pl.ds(_P, 16)]
            rowbuf[r, pl.ds(_IMG + 16, 16)] = combo_v[r, pl.ds(_P + 16, 16)]
            rowbuf[r, pl.ds(_IMG + 18, 16)] = combo_v[r, pl.ds(_P + 18, 16)]
            return c

        lax.fori_loop(0, _T, body, 0)

    def stage(j, b, combo_v, combo_n, rowbuf):
        pltpu.make_async_copy(combo_hbm.at[b], combo_v, isem).wait()

        @pl.when(b + 1 < b0 + _BPW)
        def _():
            pltpu.async_copy(combo_hbm.at[b + 1], combo_n, isem)

        @pl.when(j > 1)
        def _():
            pltpu.make_async_copy(rowbuf, out_hbm.at[b], osem).wait()

        assemble(combo_v, rowbuf)
        pltpu.async_copy(rowbuf, out_hbm.at[b], osem)

    def pair_body(j, carry):
        b = b0 + 2 * j
        stage(2 * j, b, combo0, combo1, row0)
        stage(2 * j + 1, b + 1, combo1, combo0, row1)
        return carry

    lax.fori_loop(0, _BPW // 2, pair_body, 0)
    pltpu.make_async_copy(row0, out_hbm.at[b0], osem).wait()
    pltpu.make_async_copy(row1, out_hbm.at[b0], osem).wait()


def kernel(obs_img, obs_dir, prev_action, prev_reward, done, tile_table,
           color_table, action_table, dir_W, dir_b):
    obs3 = obs_img.reshape(_B, _T, 2 * _P).astype(I32)
    ii = jnp.arange(2 * _P)[:, None]
    jj = jnp.arange(_P)[None, :]
    selt = (ii == 2 * jj).astype(F32)
    selc = (ii == 2 * jj + 1).astype(F32)
    combo, fused = _tc_setup(
        obs3, obs_dir, prev_action.reshape(_B, _T, 1).astype(I32),
        prev_reward.reshape(_B, _T, 1), done.reshape(_B, _T, 1),
        tile_table, color_table, action_table, dir_W, dir_b.reshape(1, 16),
        selt, selc)
    return _sc_encode(combo, fused.reshape(-1))


# speedup vs baseline: 1.4052x; 1.0144x over previous
import functools

import jax
import jax.numpy as jnp
from jax import lax
from jax.experimental import pallas as pl
from jax.experimental.pallas import tpu as pltpu
from jax.experimental.pallas import tpu_sc as plsc

F32 = jnp.float32
I32 = jnp.int32

_B, _T = 1024, 50
_N = _B * _T
_P = 25
_E = 32
_IMG = _P * _E
_TAIL = 34
_ROW = _IMG + _TAIL
_CW = 64
_NW = 32
_BPW = _B // _NW
_GB = 4
_TA, _TB = 24, 26
_MB = 3200


def _tc_setup_body(obs_ref, dir_ref, act_ref, rew_ref, done_ref, tile_ref,
                   color_ref, atab_ref, w_ref, b_ref, selt_ref, selc_ref,
                   combo_ref, fused_ref):
    x = obs_ref[...].astype(F32)
    t = jnp.dot(x, selt_ref[...], preferred_element_type=F32)
    c = jnp.dot(x, selc_ref[...], preferred_element_type=F32)
    combo_ref[:, 0:_P] = t * 16.0 + c

    de = jnp.dot(dir_ref[...], w_ref[...], preferred_element_type=F32) + b_ref[...]
    oh = (act_ref[...] == lax.broadcasted_iota(I32, (_MB, 16), 1)).astype(F32)
    ae = jnp.dot(oh, atab_ref[...], preferred_element_type=F32)
    combo_ref[:, _P:_P + 16] = de
    combo_ref[:, _P + 16:_P + 32] = ae
    combo_ref[:, _P + 32:_P + 33] = rew_ref[...]
    combo_ref[:, _P + 33:_P + 34] = done_ref[...]
    combo_ref[:, _P + 34:] = jnp.zeros((_MB, _CW - _P - _TAIL), F32)

    r = lax.broadcasted_iota(I32, (256, 16), 0)
    k = lax.broadcasted_iota(I32, (256, 16), 1)
    oht = ((r // 16) == k).astype(F32)
    ohc = ((r % 16) == k).astype(F32)
    fused_ref[:, 0:16] = jnp.dot(oht, tile_ref[...], preferred_element_type=F32)
    fused_ref[:, 16:32] = jnp.dot(ohc, color_ref[...], preferred_element_type=F32)


def _tc_setup(obs2, dir2, act2, rew2, done2, tile_table, color_table,
              action_table, dir_W, dir_b, selt, selc):
    grid = _N // _MB
    bspec = lambda w: pl.BlockSpec((_MB, w), lambda i: (i, 0))
    fullspec = lambda s: pl.BlockSpec(s, lambda i: (0, 0))
    return pl.pallas_call(
        _tc_setup_body,
        grid=(grid,),
        in_specs=[
            bspec(2 * _P),
            bspec(4),
            bspec(1),
            bspec(1),
            bspec(1),
            fullspec((16, 16)),
            fullspec((16, 16)),
            fullspec((16, 16)),
            fullspec((4, 16)),
            fullspec((1, 16)),
            fullspec((2 * _P, _P)),
            fullspec((2 * _P, _P)),
        ],
        out_specs=[
            bspec(_CW),
            fullspec((256, 32)),
        ],
        out_shape=[
            jax.ShapeDtypeStruct((_N, _CW), F32),
            jax.ShapeDtypeStruct((256, 32), F32),
        ],
    )(obs2, dir2, act2, rew2, done2, tile_table, color_table,
      action_table, dir_W, dir_b, selt, selc)


@functools.partial(
    pl.kernel,
    mesh=plsc.VectorSubcoreMesh(core_axis_name="c", subcore_axis_name="s"),
    compiler_params=pltpu.CompilerParams(use_tc_tiling_on_sc=True,
                                         needs_layout_passes=False),
    out_type=jax.ShapeDtypeStruct((_B, _T, _ROW), F32),
    scratch_types=[
        pltpu.VMEM((256 * _E,), F32),
        pltpu.VMEM((_GB * _T, _CW), F32),
        pltpu.VMEM((_TA, _ROW), F32),
        pltpu.VMEM((_TB, _ROW), F32),
        pltpu.SemaphoreType.DMA,
        pltpu.SemaphoreType.DMA,
    ],
)
def _sc_encode(combo_hbm, fused_hbm, out_hbm,
               fused_v, combo_v, rowa, rowb, asem, bsem):
    wid = lax.axis_index("s") * 2 + lax.axis_index("c")
    b0 = wid * _BPW
    r0 = wid * _BPW * _T
    pltpu.sync_copy(fused_hbm, fused_v)

    iota = lax.iota(I32, 16)
    dnums = lax.GatherDimensionNumbers(offset_dims=(),
                                       collapsed_slice_dims=(0,),
                                       start_index_map=(0,))

    def assemble(base, n, rowbuf):
        def body(r, c):
            pv = combo_v[base + r, pl.ds(0, 16)].astype(I32) * _E
            qv = combo_v[base + r, pl.ds(9, 16)].astype(I32) * _E
            for p0 in range(0, _P, 8):
                vals = []
                for p in range(p0, min(p0 + 8, _P)):
                    src, lane = (pv, p) if p < 16 else (qv, p - 9)
                    offv = lax.gather(
                        src, jnp.full((16, 1), lane, I32), dnums, (1,),
                        mode=lax.GatherScatterMode.PROMISE_IN_BOUNDS)
                    ia = offv + iota
                    vals.append(plsc.load_gather(fused_v, [ia]))
                    vals.append(plsc.load_gather(fused_v, [ia + 16]))
                for k, v in enumerate(vals):
                    rowbuf[r, pl.ds(_E * p0 + 16 * k, 16)] = v
            rowbuf[r, pl.ds(_IMG, 16)] = combo_v[base + r, pl.ds(_P, 16)]
            rowbuf[r, pl.ds(_IMG + 16, 16)] = combo_v[base + r, pl.ds(_P + 16, 16)]
            rowbuf[r, pl.ds(_IMG + 18, 16)] = combo_v[base + r, pl.ds(_P + 18, 16)]
            return c

        lax.fori_loop(0, n, body, 0)

    def group_body(g, carry):
        pltpu.sync_copy(combo_hbm.at[pl.ds(r0 + g * _GB * _T, _GB * _T), :],
                        combo_v)
        for jb in range(_GB):
            b = b0 + g * _GB + jb
            if jb == 0:
                @pl.when(g > 0)
                def _():
                    pltpu.make_async_copy(rowa, out_hbm.at[b, pl.ds(0, _TA)],
                                          asem).wait()
            else:
                pltpu.make_async_copy(rowa, out_hbm.at[b, pl.ds(0, _TA)],
                                      asem).wait()
            assemble(jb * _T, _TA, rowa)
            pltpu.async_copy(rowa, out_hbm.at[b, pl.ds(0, _TA)], asem)
            if jb == 0:
                @pl.when(g > 0)
                def _():
                    pltpu.make_async_copy(rowb, out_hbm.at[b, pl.ds(_TA, _TB)],
                                          bsem).wait()
            else:
                pltpu.make_async_copy(rowb, out_hbm.at[b, pl.ds(_TA, _TB)],
                                      bsem).wait()
            assemble(jb * _T + _TA, _TB, rowb)
            pltpu.async_copy(rowb, out_hbm.at[b, pl.ds(_TA, _TB)], bsem)
        return carry

    lax.fori_loop(0, _BPW // _GB, group_body, 0)
    pltpu.make_async_copy(rowa, out_hbm.at[b0, pl.ds(0, _TA)], asem).wait()
    pltpu.make_async_copy(rowb, out_hbm.at[b0, pl.ds(_TA, _TB)], bsem).wait()


def kernel(obs_img, obs_dir, prev_action, prev_reward, done, tile_table,
           color_table, action_table, dir_W, dir_b):
    obs2 = obs_img.reshape(_N, 2 * _P).astype(I32)
    ii = jnp.arange(2 * _P)[:, None]
    jj = jnp.arange(_P)[None, :]
    selt = (ii == 2 * jj).astype(F32)
    selc = (ii == 2 * jj + 1).astype(F32)
    combo, fused = _tc_setup(
        obs2, obs_dir.reshape(_N, 4), prev_action.reshape(_N, 1).astype(I32),
        prev_reward.reshape(_N, 1), done.reshape(_N, 1),
        tile_table, color_table, action_table, dir_W, dir_b.reshape(1, 16),
        selt, selc)
    return _sc_encode(combo, fused.reshape(-1))
